# baseline (speedup 1.0000x reference)
import math
import functools

import jax
import jax.numpy as jnp
from jax import lax
from jax.experimental import pallas as pl
from jax.experimental.pallas import tpu as pltpu
from jax.experimental.pallas import tpu_sc as plsc

N = 10000
B = 64
H = 8
DV = 16
DC = 128
DE = 124
HD = H * DV

NP = 10240
TN = 1024
T = NP // TN
NEG = -1e30

NSC = 2
NSUB = 16
NW = NSC * NSUB
RPW = NP // NW
CHUNK = 80
NCHUNK = RPW // CHUNK
ACCR = 72


def _head_sum_matrix():
    lane = lax.broadcasted_iota(jnp.int32, (HD, H), 0)
    head = lax.broadcasted_iota(jnp.int32, (HD, H), 1)
    return (lane // DV == head).astype(jnp.float32)


def _one_hot(b_ref):
    batch_col = b_ref[:, 0:1]
    iota_b = lax.broadcasted_iota(jnp.int32, (TN, B), 1)
    maskb = batch_col == iota_b
    return maskb, maskb.astype(jnp.float32)


def _tc_scores_body(x_ref, b_ref, wk_ref, ctx_ref, wq_ref,
                    u_ref, segmax_ref, qflat_ref, qflat_s, segmax_s):
    i = pl.program_id(0)
    S = _head_sum_matrix()

    @pl.when(i == 0)
    def _init():
        qflat_s[:, :] = jnp.dot(ctx_ref[:, :], wq_ref[:, :],
                                preferred_element_type=jnp.float32)
        segmax_s[:, :] = jnp.full((B, H), NEG, jnp.float32)

    maskb, maskf = _one_hot(b_ref)
    k = jnp.dot(x_ref[:, :], wk_ref[:, :], preferred_element_type=jnp.float32)
    qg = jnp.dot(maskf, qflat_s[:, :], preferred_element_type=jnp.float32)
    u = jnp.dot(k * qg, S,
                preferred_element_type=jnp.float32) * (1.0 / math.sqrt(DV))
    u_ref[:, :] = u
    for h in range(H):
        col = u[:, h:h + 1]
        masked = jnp.where(maskb, col, NEG)
        tmax = jnp.max(masked, axis=0)
        cur = segmax_s[:, h]
        segmax_s[:, h] = jnp.maximum(cur, tmax)

    @pl.when(i == T - 1)
    def _emit():
        segmax_ref[:, :] = segmax_s[:, :]
        qflat_ref[:, :] = qflat_s[:, :]


@jax.jit
def _tc_scores(x, batch2d, wk, context, wq):
    return pl.pallas_call(
        _tc_scores_body,
        grid=(T,),
        in_specs=[
            pl.BlockSpec((TN, DC), lambda i: (i, 0)),
            pl.BlockSpec((TN, 1), lambda i: (i, 0)),
            pl.BlockSpec((DC, HD), lambda i: (0, 0)),
            pl.BlockSpec((B, DC), lambda i: (0, 0)),
            pl.BlockSpec((DC, HD), lambda i: (0, 0)),
        ],
        out_specs=[
            pl.BlockSpec((TN, H), lambda i: (i, 0)),
            pl.BlockSpec((B, H), lambda i: (0, 0)),
            pl.BlockSpec((B, HD), lambda i: (0, 0)),
        ],
        out_shape=[
            jax.ShapeDtypeStruct((NP, H), jnp.float32),
            jax.ShapeDtypeStruct((B, H), jnp.float32),
            jax.ShapeDtypeStruct((B, HD), jnp.float32),
        ],
        scratch_shapes=[
            pltpu.VMEM((B, HD), jnp.float32),
            pltpu.VMEM((B, H), jnp.float32),
        ],
    )(x, batch2d, wk, context, wq)


def _tc_weights_body(x_ref, b_ref, u_ref, segmax_ref, wv_ref,
                     w_ref, ssum_ref, segsum_s):
    i = pl.program_id(0)
    S = _head_sum_matrix()

    @pl.when(i == 0)
    def _init():
        segsum_s[:, :] = jnp.zeros((B, H), jnp.float32)

    _, maskf = _one_hot(b_ref)
    v = jnp.dot(x_ref[:, :], wv_ref[:, :], preferred_element_type=jnp.float32)
    gmax = jnp.dot(maskf, segmax_ref[:, :],
                   preferred_element_type=jnp.float32)
    e = jnp.exp(u_ref[:, :] - gmax)
    e_exp = jnp.dot(e, S.T, preferred_element_type=jnp.float32)
    w_ref[:, :] = e_exp * v
    contract0 = (((0,), (0,)), ((), ()))
    segsum_s[:, :] += lax.dot_general(
        maskf, e, contract0, preferred_element_type=jnp.float32)

    @pl.when(i == T - 1)
    def _emit():
        ssum_ref[:, :] = segsum_s[:, :]


@jax.jit
def _tc_weights(x, batch2d, u, segmax, wv):
    return pl.pallas_call(
        _tc_weights_body,
        grid=(T,),
        in_specs=[
            pl.BlockSpec((TN, DC), lambda i: (i, 0)),
            pl.BlockSpec((TN, 1), lambda i: (i, 0)),
            pl.BlockSpec((TN, H), lambda i: (i, 0)),
            pl.BlockSpec((B, H), lambda i: (0, 0)),
            pl.BlockSpec((DC, HD), lambda i: (0, 0)),
        ],
        out_specs=[
            pl.BlockSpec((TN, HD), lambda i: (i, 0)),
            pl.BlockSpec((B, H), lambda i: (0, 0)),
        ],
        out_shape=[
            jax.ShapeDtypeStruct((NP, HD), jnp.float32),
            jax.ShapeDtypeStruct((B, H), jnp.float32),
        ],
        scratch_shapes=[
            pltpu.VMEM((B, H), jnp.float32),
        ],
    )(x, batch2d, u, segmax, wv)


def _sc_scatter_body(w_hbm, batch_hbm, zeros_hbm, out_hbm, idx_v, w_v, acc_sh):
    c = lax.axis_index("c")
    s = lax.axis_index("s")
    wid = s * NSC + c
    base = wid * RPW

    @pl.when(s == 0)
    def _init():
        pltpu.sync_copy(zeros_hbm, acc_sh)
    plsc.subcore_barrier()

    for j in range(NCHUNK):
        pltpu.sync_copy(batch_hbm.at[pl.ds(base + j * CHUNK, CHUNK)],
                        idx_v.at[j])
    pltpu.sync_copy(w_hbm.at[pl.ds(base, RPW)], w_v)
    for j in range(NCHUNK):
        pltpu.sync_copy(w_v.at[pl.ds(j * CHUNK, CHUNK)],
                        acc_sh.at[idx_v.at[j]], add=True)

    plsc.subcore_barrier()

    @pl.when(s == 0)
    def _emit():
        pltpu.sync_copy(acc_sh, out_hbm.at[c])


@functools.cache
def _sc_scatter():
    mesh = plsc.VectorSubcoreMesh(core_axis_name="c", subcore_axis_name="s",
                                  num_cores=NSC, num_subcores=NSUB)
    return pl.kernel(
        _sc_scatter_body,
        out_type=jax.ShapeDtypeStruct((NSC, ACCR, HD), jnp.float32),
        mesh=mesh,
        scratch_types=[
            pltpu.VMEM((NCHUNK, CHUNK), jnp.int32),
            pltpu.VMEM((RPW, HD), jnp.float32),
            pltpu.VMEM_SHARED((ACCR, HD), jnp.float32),
        ],
    )


def _tc_finish_body(a0_ref, a1_ref, ssum_ref, qflat_ref, qc_ref, pf_ref,
                    out_ref):
    S = _head_sum_matrix()
    lane2 = lax.broadcasted_iota(jnp.int32, (HD, DV), 0)
    vpos = lax.broadcasted_iota(jnp.int32, (HD, DV), 1)
    R = (lax.rem(lane2, DV) == vpos).astype(jnp.float32)

    acc = a0_ref[:, :] + a1_ref[:, :]
    agg_raw = acc[0:B, :]
    ssum_exp = jnp.dot(ssum_ref[:, :], S.T,
                       preferred_element_type=jnp.float32)
    agg = agg_raw / (ssum_exp + 1e-16)
    hf = qc_ref[0, 0] * qflat_ref[:, :] + agg
    hsum = jnp.dot(hf, R, preferred_element_type=jnp.float32)
    out_ref[:, :] = jnp.dot(hsum, pf_ref[:, :],
                            preferred_element_type=jnp.float32)


@jax.jit
def _tc_finish(a0, a1, ssum, qflat, qcb, pf):
    return pl.pallas_call(
        _tc_finish_body,
        out_shape=jax.ShapeDtypeStruct((B, DE), jnp.float32),
    )(a0, a1, ssum, qflat, qcb, pf)


def kernel(x, edge_index, batch, context, proj_query, proj_keys, proj_values,
           query_coef, proj_final):
    wk = proj_keys.transpose(1, 0, 2).reshape(DC, HD)
    wv = proj_values.transpose(1, 0, 2).reshape(DC, HD)
    wq = proj_query.transpose(1, 0, 2).reshape(DC, HD)
    x_p = jnp.pad(x, ((0, NP - N), (0, 0)))
    batch_p = jnp.pad(batch, (0, NP - N), constant_values=B)
    batch2d = batch_p.reshape(NP, 1)
    qcb = jnp.broadcast_to(query_coef.reshape(1, 1), (8, 128))
    zeros = jnp.zeros((ACCR, HD), jnp.float32)

    u, segmax, qflat = _tc_scores(x_p, batch2d, wk, context, wq)
    w_rows, ssum = _tc_weights(x_p, batch2d, u, segmax, wv)
    acc = _sc_scatter()(w_rows, batch_p, zeros)
    out = _tc_finish(acc[0], acc[1], ssum, qflat, qcb, proj_final)
    return out


# speedup vs baseline: 1.2496x; 1.2496x over previous
import math
import functools

import jax
import jax.numpy as jnp
from jax import lax
from jax.experimental import pallas as pl
from jax.experimental.pallas import tpu as pltpu
from jax.experimental.pallas import tpu_sc as plsc

N = 10000
B = 64
H = 8
DV = 16
DC = 128
DE = 124
HD = H * DV

NP = 10240
TN = 1024
T = NP // TN
NEG = -1e30

NSC = 2
NSUB = 16
NW = NSC * NSUB
RPW = NP // NW
CHUNK = 80
NCHUNK = RPW // CHUNK
ACCR = 72


def _head_sum_matrix():
    lane = lax.broadcasted_iota(jnp.int32, (HD, H), 0)
    head = lax.broadcasted_iota(jnp.int32, (HD, H), 1)
    return (lane // DV == head).astype(jnp.float32)


def _tc_main_body(x_ref, b_ref, wk_ref, wv_ref, ctx_ref, wq_ref,
                  w_ref, ssum_ref, qflat_ref,
                  u_s, umax_s, segsum_s, qflat_s):
    i = pl.program_id(0)
    t = lax.rem(i, T)
    S = _head_sum_matrix()

    @pl.when(i == 0)
    def _init():
        qflat_s[:, :] = jnp.dot(ctx_ref[:, :], wq_ref[:, :],
                                preferred_element_type=jnp.float32)
        umax_s[:, :] = jnp.full((1, H), NEG, jnp.float32)
        segsum_s[:, :] = jnp.zeros((B, H), jnp.float32)

    batch_col = b_ref[:, 0:1]
    iota_b = lax.broadcasted_iota(jnp.int32, (TN, B), 1)
    maskf = (batch_col == iota_b).astype(jnp.float32)

    @pl.when(i < T)
    def _phase1():
        k = jnp.dot(x_ref[:, :], wk_ref[:, :],
                    preferred_element_type=jnp.float32)
        qg = jnp.dot(maskf, qflat_s[:, :],
                     preferred_element_type=jnp.float32)
        u = jnp.dot(k * qg, S,
                    preferred_element_type=jnp.float32) * (1.0 / math.sqrt(DV))
        u_s[pl.ds(t * TN, TN), :] = u
        tmax = jnp.max(u, axis=0, keepdims=True)
        umax_s[:, :] = jnp.maximum(umax_s[:, :], tmax)

    @pl.when(i >= T)
    def _phase2():
        v = jnp.dot(x_ref[:, :], wv_ref[:, :],
                    preferred_element_type=jnp.float32)
        u = u_s[pl.ds(t * TN, TN), :]
        e = jnp.exp(u - umax_s[:, :])
        e_exp = jnp.dot(e, S.T, preferred_element_type=jnp.float32)
        w_ref[:, :] = e_exp * v
        contract0 = (((0,), (0,)), ((), ()))
        segsum_s[:, :] += lax.dot_general(
            maskf, e, contract0, preferred_element_type=jnp.float32)

        @pl.when(i == 2 * T - 1)
        def _emit():
            ssum_ref[:, :] = segsum_s[:, :]
            qflat_ref[:, :] = qflat_s[:, :]


@jax.jit
def _tc_main(x, batch2d, wk, wv, context, wq):
    return pl.pallas_call(
        _tc_main_body,
        grid=(2 * T,),
        in_specs=[
            pl.BlockSpec((TN, DC), lambda i: (lax.rem(i, T), 0)),
            pl.BlockSpec((TN, 1), lambda i: (lax.rem(i, T), 0)),
            pl.BlockSpec((DC, HD), lambda i: (0, 0)),
            pl.BlockSpec((DC, HD), lambda i: (0, 0)),
            pl.BlockSpec((B, DC), lambda i: (0, 0)),
            pl.BlockSpec((DC, HD), lambda i: (0, 0)),
        ],
        out_specs=[
            pl.BlockSpec((TN, HD), lambda i: (jnp.maximum(i - T, 0), 0)),
            pl.BlockSpec((B, H), lambda i: (0, 0)),
            pl.BlockSpec((B, HD), lambda i: (0, 0)),
        ],
        out_shape=[
            jax.ShapeDtypeStruct((NP, HD), jnp.float32),
            jax.ShapeDtypeStruct((B, H), jnp.float32),
            jax.ShapeDtypeStruct((B, HD), jnp.float32),
        ],
        scratch_shapes=[
            pltpu.VMEM((NP, H), jnp.float32),
            pltpu.VMEM((1, H), jnp.float32),
            pltpu.VMEM((B, H), jnp.float32),
            pltpu.VMEM((B, HD), jnp.float32),
        ],
    )(x, batch2d, wk, wv, context, wq)


def _sc_scatter_body(w_hbm, batch_hbm, zeros_hbm, out_hbm, idx_v, w_v, acc_sh):
    c = lax.axis_index("c")
    s = lax.axis_index("s")
    wid = s * NSC + c
    base = wid * RPW

    @pl.when(s == 0)
    def _init():
        pltpu.sync_copy(zeros_hbm, acc_sh)
    plsc.subcore_barrier()

    for j in range(NCHUNK):
        pltpu.sync_copy(batch_hbm.at[pl.ds(base + j * CHUNK, CHUNK)],
                        idx_v.at[j])
    pltpu.sync_copy(w_hbm.at[pl.ds(base, RPW)], w_v)
    for j in range(NCHUNK):
        pltpu.sync_copy(w_v.at[pl.ds(j * CHUNK, CHUNK)],
                        acc_sh.at[idx_v.at[j]], add=True)

    plsc.subcore_barrier()

    @pl.when(s == 0)
    def _emit():
        pltpu.sync_copy(acc_sh, out_hbm.at[c])


@functools.cache
def _sc_scatter():
    mesh = plsc.VectorSubcoreMesh(core_axis_name="c", subcore_axis_name="s",
                                  num_cores=NSC, num_subcores=NSUB)
    return pl.kernel(
        _sc_scatter_body,
        out_type=jax.ShapeDtypeStruct((NSC, ACCR, HD), jnp.float32),
        mesh=mesh,
        scratch_types=[
            pltpu.VMEM((NCHUNK, CHUNK), jnp.int32),
            pltpu.VMEM((RPW, HD), jnp.float32),
            pltpu.VMEM_SHARED((ACCR, HD), jnp.float32),
        ],
    )


def _tc_finish_body(a0_ref, a1_ref, ssum_ref, qflat_ref, qc_ref, pf_ref,
                    out_ref):
    S = _head_sum_matrix()
    lane2 = lax.broadcasted_iota(jnp.int32, (HD, DV), 0)
    vpos = lax.broadcasted_iota(jnp.int32, (HD, DV), 1)
    R = (lax.rem(lane2, DV) == vpos).astype(jnp.float32)

    acc = a0_ref[:, :] + a1_ref[:, :]
    agg_raw = acc[0:B, :]
    ssum_exp = jnp.dot(ssum_ref[:, :], S.T,
                       preferred_element_type=jnp.float32)
    agg = agg_raw / (ssum_exp + 1e-16)
    hf = qc_ref[0, 0] * qflat_ref[:, :] + agg
    hsum = jnp.dot(hf, R, preferred_element_type=jnp.float32)
    out_ref[:, :] = jnp.dot(hsum, pf_ref[:, :],
                            preferred_element_type=jnp.float32)


@jax.jit
def _tc_finish(a0, a1, ssum, qflat, qcb, pf):
    return pl.pallas_call(
        _tc_finish_body,
        out_shape=jax.ShapeDtypeStruct((B, DE), jnp.float32),
    )(a0, a1, ssum, qflat, qcb, pf)


def kernel(x, edge_index, batch, context, proj_query, proj_keys, proj_values,
           query_coef, proj_final):
    wk = proj_keys.transpose(1, 0, 2).reshape(DC, HD)
    wv = proj_values.transpose(1, 0, 2).reshape(DC, HD)
    wq = proj_query.transpose(1, 0, 2).reshape(DC, HD)
    x_p = jnp.pad(x, ((0, NP - N), (0, 0)))
    batch_p = jnp.pad(batch, (0, NP - N), constant_values=B)
    batch2d = batch_p.reshape(NP, 1)
    qcb = jnp.broadcast_to(query_coef.reshape(1, 1), (8, 128))
    zeros = jnp.zeros((ACCR, HD), jnp.float32)

    w_rows, ssum, qflat = _tc_main(x_p, batch2d, wk, wv, context, wq)
    acc = _sc_scatter()(w_rows, batch_p, zeros)
    out = _tc_finish(acc[0], acc[1], ssum, qflat, qcb, proj_final)
    return out


# speedup vs baseline: 1.3104x; 1.0487x over previous
import math
import functools

import jax
import jax.numpy as jnp
from jax import lax
from jax.experimental import pallas as pl
from jax.experimental.pallas import tpu as pltpu
from jax.experimental.pallas import tpu_sc as plsc

N = 10000
B = 64
H = 8
DV = 16
DC = 128
DE = 124
HD = H * DV

NP = 10240
TN = 1024
T = NP // TN
NEG = -1e30

NSC = 2
NSUB = 16
NW = NSC * NSUB
RPW = NP // NW
CHUNK = 80
NCHUNK = RPW // CHUNK
ACCR = 72


def _head_sum_matrix():
    lane = lax.broadcasted_iota(jnp.int32, (HD, H), 0)
    head = lax.broadcasted_iota(jnp.int32, (HD, H), 1)
    return (lane // DV == head).astype(jnp.float32)


def _tc_main_body(x_ref, b_ref, wk_ref, wv_ref, ctx_ref, wq_ref,
                  w_ref, ssum_ref, qflat_ref,
                  u_s, umax_s, segsum_s, qflat_s):
    i = pl.program_id(0)
    t = lax.rem(i, T)
    S = _head_sum_matrix()

    @pl.when(i == 0)
    def _init():
        qflat_s[:, :] = jnp.dot(ctx_ref[:, :], wq_ref[:, :],
                                preferred_element_type=jnp.float32)
        umax_s[:, :] = jnp.full((1, H), NEG, jnp.float32)
        segsum_s[:, :] = jnp.zeros((B, H), jnp.float32)

    batch_col = b_ref[:, 0:1]
    iota_b = lax.broadcasted_iota(jnp.int32, (TN, B), 1)
    maskf = (batch_col == iota_b).astype(jnp.float32)

    @pl.when(i < T)
    def _phase1():
        k = jnp.dot(x_ref[:, :], wk_ref[:, :],
                    preferred_element_type=jnp.float32)
        qg = jnp.dot(maskf, qflat_s[:, :],
                     preferred_element_type=jnp.float32)
        u = jnp.dot(k * qg, S,
                    preferred_element_type=jnp.float32) * (1.0 / math.sqrt(DV))
        u_s[pl.ds(t * TN, TN), :] = u
        valid = batch_col != B
        u_m = jnp.where(valid, u, NEG)
        tmax = jnp.max(u_m, axis=0, keepdims=True)
        umax_s[:, :] = jnp.maximum(umax_s[:, :], tmax)

    @pl.when(i >= T)
    def _phase2():
        v = jnp.dot(x_ref[:, :], wv_ref[:, :],
                    preferred_element_type=jnp.float32)
        u = u_s[pl.ds(t * TN, TN), :]
        e = jnp.exp(u - umax_s[:, :])
        e_exp = jnp.dot(e, S.T, preferred_element_type=jnp.float32)
        w_ref[:, :] = e_exp * v
        contract0 = (((0,), (0,)), ((), ()))
        segsum_s[:, :] += lax.dot_general(
            maskf, e, contract0, preferred_element_type=jnp.float32)

        @pl.when(i == 2 * T - 1)
        def _emit():
            ssum_ref[:, :] = segsum_s[:, :]
            qflat_ref[:, :] = qflat_s[:, :]


@jax.jit
def _tc_main(x, batch2d, wk, wv, context, wq):
    return pl.pallas_call(
        _tc_main_body,
        grid=(2 * T,),
        in_specs=[
            pl.BlockSpec((TN, DC), lambda i: (lax.rem(i, T), 0)),
            pl.BlockSpec((TN, 1), lambda i: (lax.rem(i, T), 0)),
            pl.BlockSpec((DC, HD), lambda i: (0, 0)),
            pl.BlockSpec((DC, HD), lambda i: (0, 0)),
            pl.BlockSpec((B, DC), lambda i: (0, 0)),
            pl.BlockSpec((DC, HD), lambda i: (0, 0)),
        ],
        out_specs=[
            pl.BlockSpec((TN, HD), lambda i: (jnp.maximum(i - T, 0), 0)),
            pl.BlockSpec((B, H), lambda i: (0, 0)),
            pl.BlockSpec((B, HD), lambda i: (0, 0)),
        ],
        out_shape=[
            jax.ShapeDtypeStruct((NP, HD), jnp.float32),
            jax.ShapeDtypeStruct((B, H), jnp.float32),
            jax.ShapeDtypeStruct((B, HD), jnp.float32),
        ],
        scratch_shapes=[
            pltpu.VMEM((NP, H), jnp.float32),
            pltpu.VMEM((1, H), jnp.float32),
            pltpu.VMEM((B, H), jnp.float32),
            pltpu.VMEM((B, HD), jnp.float32),
        ],
    )(x, batch2d, wk, wv, context, wq)


def _sc_scatter_body(w_hbm, batch_hbm, zeros_hbm, out_hbm, idx_v, w_v, acc_sh):
    c = lax.axis_index("c")
    s = lax.axis_index("s")
    wid = s * NSC + c
    base = wid * RPW

    @pl.when(s == 0)
    def _init():
        pltpu.sync_copy(zeros_hbm, acc_sh)
    plsc.subcore_barrier()

    for j in range(NCHUNK):
        pltpu.sync_copy(batch_hbm.at[pl.ds(base + j * CHUNK, CHUNK)],
                        idx_v.at[j])
    pltpu.sync_copy(w_hbm.at[pl.ds(base, RPW)], w_v)
    for j in range(NCHUNK):
        pltpu.sync_copy(w_v.at[pl.ds(j * CHUNK, CHUNK)],
                        acc_sh.at[idx_v.at[j]], add=True)

    plsc.subcore_barrier()

    @pl.when(s == 0)
    def _emit():
        pltpu.sync_copy(acc_sh, out_hbm.at[c])


@functools.cache
def _sc_scatter():
    mesh = plsc.VectorSubcoreMesh(core_axis_name="c", subcore_axis_name="s",
                                  num_cores=NSC, num_subcores=NSUB)
    return pl.kernel(
        _sc_scatter_body,
        out_type=jax.ShapeDtypeStruct((NSC, ACCR, HD), jnp.float32),
        mesh=mesh,
        scratch_types=[
            pltpu.VMEM((NCHUNK, CHUNK), jnp.int32),
            pltpu.VMEM((RPW, HD), jnp.float32),
            pltpu.VMEM_SHARED((ACCR, HD), jnp.float32),
        ],
    )


def _tc_finish_body(a0_ref, a1_ref, ssum_ref, qflat_ref, qc_ref, pf_ref,
                    out_ref):
    S = _head_sum_matrix()
    lane2 = lax.broadcasted_iota(jnp.int32, (HD, DV), 0)
    vpos = lax.broadcasted_iota(jnp.int32, (HD, DV), 1)
    R = (lax.rem(lane2, DV) == vpos).astype(jnp.float32)

    acc = a0_ref[:, :] + a1_ref[:, :]
    agg_raw = acc[0:B, :]
    ssum_exp = jnp.dot(ssum_ref[:, :], S.T,
                       preferred_element_type=jnp.float32)
    agg = agg_raw / (ssum_exp + 1e-16)
    hf = qc_ref[0, 0] * qflat_ref[:, :] + agg
    hsum = jnp.dot(hf, R, preferred_element_type=jnp.float32)
    out_ref[:, :] = jnp.dot(hsum, pf_ref[:, :],
                            preferred_element_type=jnp.float32)


@jax.jit
def _tc_finish(a0, a1, ssum, qflat, qcb, pf):
    return pl.pallas_call(
        _tc_finish_body,
        out_shape=jax.ShapeDtypeStruct((B, DE), jnp.float32),
    )(a0, a1, ssum, qflat, qcb, pf)


def kernel(x, edge_index, batch, context, proj_query, proj_keys, proj_values,
           query_coef, proj_final):
    wk = proj_keys.transpose(1, 0, 2).reshape(DC, HD)
    wv = proj_values.transpose(1, 0, 2).reshape(DC, HD)
    wq = proj_query.transpose(1, 0, 2).reshape(DC, HD)
    batch_p = jnp.pad(batch, (0, NP - N), constant_values=B)
    batch2d = batch_p.reshape(NP, 1)
    qcb = jnp.broadcast_to(query_coef.reshape(1, 1), (8, 128))
    zeros = jnp.zeros((ACCR, HD), jnp.float32)

    w_rows, ssum, qflat = _tc_main(x, batch2d, wk, wv, context, wq)
    acc = _sc_scatter()(w_rows, batch_p, zeros)
    out = _tc_finish(acc[0], acc[1], ssum, qflat, qcb, proj_final)
    return out


# speedup vs baseline: 1.4567x; 1.1116x over previous
import math
import functools

import jax
import jax.numpy as jnp
from jax import lax
from jax.experimental import pallas as pl
from jax.experimental.pallas import tpu as pltpu
from jax.experimental.pallas import tpu_sc as plsc

N = 10000
B = 64
H = 8
DV = 16
DC = 128
DE = 124
HD = H * DV

NP = 10240
TN = 2048
T = NP // TN
NEG = -1e30

NSC = 2
NSUB = 16
NW = NSC * NSUB
RPW = NP // NW
CHUNK = 80
NCHUNK = RPW // CHUNK
ACCR = 72


def _head_sum_matrix():
    lane = lax.broadcasted_iota(jnp.int32, (HD, H), 0)
    head = lax.broadcasted_iota(jnp.int32, (HD, H), 1)
    return (lane // DV == head).astype(jnp.float32)


def _tc_main_body(x_ref, b_ref, wk_ref, wv_ref, ctx_ref, wq_ref,
                  w_ref, ssum_ref, qflat_ref,
                  u_s, umax_s, segsum_s, qflat_s):
    i = pl.program_id(0)
    t = lax.rem(i, T)
    S = _head_sum_matrix()

    @pl.when(i == 0)
    def _init():
        qflat_s[:, :] = jnp.dot(ctx_ref[:, :], wq_ref[:, :],
                                preferred_element_type=jnp.float32)
        umax_s[:, :] = jnp.full((1, H), NEG, jnp.float32)
        segsum_s[:, :] = jnp.zeros((B, H), jnp.float32)

    batch_col = b_ref[:, 0:1]
    iota_b = lax.broadcasted_iota(jnp.int32, (TN, B), 1)
    maskf = (batch_col == iota_b).astype(jnp.float32)

    @pl.when(i < T)
    def _phase1():
        k = jnp.dot(x_ref[:, :], wk_ref[:, :],
                    preferred_element_type=jnp.float32)
        qg = jnp.dot(maskf, qflat_s[:, :],
                     preferred_element_type=jnp.float32)
        u = jnp.dot(k * qg, S,
                    preferred_element_type=jnp.float32) * (1.0 / math.sqrt(DV))
        u_s[pl.ds(t * TN, TN), :] = u
        valid = batch_col != B
        u_m = jnp.where(valid, u, NEG)
        tmax = jnp.max(u_m, axis=0, keepdims=True)
        umax_s[:, :] = jnp.maximum(umax_s[:, :], tmax)

    @pl.when(i >= T)
    def _phase2():
        v = jnp.dot(x_ref[:, :], wv_ref[:, :],
                    preferred_element_type=jnp.float32)
        u = u_s[pl.ds(t * TN, TN), :]
        e = jnp.exp(u - umax_s[:, :])
        e_exp = jnp.dot(e, S.T, preferred_element_type=jnp.float32)
        w_ref[:, :] = e_exp * v
        contract0 = (((0,), (0,)), ((), ()))
        segsum_s[:, :] += lax.dot_general(
            maskf, e, contract0, preferred_element_type=jnp.float32)

        @pl.when(i == 2 * T - 1)
        def _emit():
            ssum_ref[:, :] = segsum_s[:, :]
            qflat_ref[:, :] = qflat_s[:, :]


@jax.jit
def _tc_main(x, batch2d, wk, wv, context, wq):
    return pl.pallas_call(
        _tc_main_body,
        grid=(2 * T,),
        in_specs=[
            pl.BlockSpec((TN, DC), lambda i: (lax.rem(i, T), 0)),
            pl.BlockSpec((TN, 1), lambda i: (lax.rem(i, T), 0)),
            pl.BlockSpec((DC, HD), lambda i: (0, 0)),
            pl.BlockSpec((DC, HD), lambda i: (0, 0)),
            pl.BlockSpec((B, DC), lambda i: (0, 0)),
            pl.BlockSpec((DC, HD), lambda i: (0, 0)),
        ],
        out_specs=[
            pl.BlockSpec((TN, HD), lambda i: (jnp.maximum(i - T, 0), 0)),
            pl.BlockSpec((B, H), lambda i: (0, 0)),
            pl.BlockSpec((B, HD), lambda i: (0, 0)),
        ],
        out_shape=[
            jax.ShapeDtypeStruct((NP, HD), jnp.float32),
            jax.ShapeDtypeStruct((B, H), jnp.float32),
            jax.ShapeDtypeStruct((B, HD), jnp.float32),
        ],
        scratch_shapes=[
            pltpu.VMEM((NP, H), jnp.float32),
            pltpu.VMEM((1, H), jnp.float32),
            pltpu.VMEM((B, H), jnp.float32),
            pltpu.VMEM((B, HD), jnp.float32),
        ],
    )(x, batch2d, wk, wv, context, wq)


def _sc_scatter_body(w_hbm, batch_hbm, zeros_hbm, out_hbm, idx_v, w_v, acc_sh):
    c = lax.axis_index("c")
    s = lax.axis_index("s")
    wid = s * NSC + c
    base = wid * RPW

    @pl.when(s == 0)
    def _init():
        pltpu.sync_copy(zeros_hbm, acc_sh)
    plsc.subcore_barrier()

    for j in range(NCHUNK):
        pltpu.sync_copy(batch_hbm.at[pl.ds(base + j * CHUNK, CHUNK)],
                        idx_v.at[j])
    pltpu.sync_copy(w_hbm.at[pl.ds(base, RPW)], w_v)
    for j in range(NCHUNK):
        pltpu.sync_copy(w_v.at[pl.ds(j * CHUNK, CHUNK)],
                        acc_sh.at[idx_v.at[j]], add=True)

    plsc.subcore_barrier()

    @pl.when(s == 0)
    def _emit():
        pltpu.sync_copy(acc_sh, out_hbm.at[c])


@functools.cache
def _sc_scatter():
    mesh = plsc.VectorSubcoreMesh(core_axis_name="c", subcore_axis_name="s",
                                  num_cores=NSC, num_subcores=NSUB)
    return pl.kernel(
        _sc_scatter_body,
        out_type=jax.ShapeDtypeStruct((NSC, ACCR, HD), jnp.float32),
        mesh=mesh,
        scratch_types=[
            pltpu.VMEM((NCHUNK, CHUNK), jnp.int32),
            pltpu.VMEM((RPW, HD), jnp.float32),
            pltpu.VMEM_SHARED((ACCR, HD), jnp.float32),
        ],
    )


def _tc_finish_body(a0_ref, a1_ref, ssum_ref, qflat_ref, qc_ref, pf_ref,
                    out_ref):
    S = _head_sum_matrix()
    lane2 = lax.broadcasted_iota(jnp.int32, (HD, DV), 0)
    vpos = lax.broadcasted_iota(jnp.int32, (HD, DV), 1)
    R = (lax.rem(lane2, DV) == vpos).astype(jnp.float32)

    acc = a0_ref[:, :] + a1_ref[:, :]
    agg_raw = acc[0:B, :]
    ssum_exp = jnp.dot(ssum_ref[:, :], S.T,
                       preferred_element_type=jnp.float32)
    agg = agg_raw / (ssum_exp + 1e-16)
    hf = qc_ref[0, 0] * qflat_ref[:, :] + agg
    hsum = jnp.dot(hf, R, preferred_element_type=jnp.float32)
    out_ref[:, :] = jnp.dot(hsum, pf_ref[:, :],
                            preferred_element_type=jnp.float32)


@jax.jit
def _tc_finish(a0, a1, ssum, qflat, qcb, pf):
    return pl.pallas_call(
        _tc_finish_body,
        out_shape=jax.ShapeDtypeStruct((B, DE), jnp.float32),
    )(a0, a1, ssum, qflat, qcb, pf)


def kernel(x, edge_index, batch, context, proj_query, proj_keys, proj_values,
           query_coef, proj_final):
    wk = proj_keys.transpose(1, 0, 2).reshape(DC, HD)
    wv = proj_values.transpose(1, 0, 2).reshape(DC, HD)
    wq = proj_query.transpose(1, 0, 2).reshape(DC, HD)
    batch_p = jnp.pad(batch, (0, NP - N), constant_values=B)
    batch2d = batch_p.reshape(NP, 1)
    qcb = jnp.broadcast_to(query_coef.reshape(1, 1), (8, 128))
    zeros = jnp.zeros((ACCR, HD), jnp.float32)

    w_rows, ssum, qflat = _tc_main(x, batch2d, wk, wv, context, wq)
    acc = _sc_scatter()(w_rows, batch_p, zeros)
    out = _tc_finish(acc[0], acc[1], ssum, qflat, qcb, proj_final)
    return out


# speedup vs baseline: 1.5220x; 1.0448x over previous
import math
import functools

import jax
import jax.numpy as jnp
from jax import lax
from jax.experimental import pallas as pl
from jax.experimental.pallas import tpu as pltpu
from jax.experimental.pallas import tpu_sc as plsc

N = 10000
B = 64
H = 8
DV = 16
DC = 128
DE = 124
HD = H * DV

NP = 10240
TN = 2048
T = NP // TN
NEG = -1e30

NSC = 2
NSUB = 16
NW = NSC * NSUB
RPW = NP // NW
CHUNK = 80
NCHUNK = RPW // CHUNK
ACCR = 72


def _head_sum_matrix():
    lane = lax.broadcasted_iota(jnp.int32, (HD, H), 0)
    head = lax.broadcasted_iota(jnp.int32, (HD, H), 1)
    return (lane // DV == head).astype(jnp.float32)


def _tc_main_body(x_ref, b_ref, wk_ref, wv_ref, ctx_ref, wq_ref,
                  w_ref, ssum_ref, qflat_ref,
                  u_s, umax_s, segsum_s, qflat_s, wk_s, wv_s):
    i = pl.program_id(0)
    t = lax.rem(i, T)
    S = _head_sum_matrix()

    @pl.when(i == 0)
    def _init():
        for h in range(H):
            sl = pl.ds(h * DV, DV)
            wk_s[:, sl] = wk_ref[h]
            wv_s[:, sl] = wv_ref[h]
            qflat_s[:, sl] = jnp.dot(ctx_ref[:, :], wq_ref[h],
                                     preferred_element_type=jnp.float32)
        umax_s[:, :] = jnp.full((1, H), NEG, jnp.float32)
        segsum_s[:, :] = jnp.zeros((B, H), jnp.float32)

    batch_col = b_ref[:, 0:1]
    iota_b = lax.broadcasted_iota(jnp.int32, (TN, B), 1)
    maskf = (batch_col == iota_b).astype(jnp.float32)

    @pl.when(i < T)
    def _phase1():
        k = jnp.dot(x_ref[:, :], wk_s[:, :],
                    preferred_element_type=jnp.float32)
        qg = jnp.dot(maskf, qflat_s[:, :],
                     preferred_element_type=jnp.float32)
        u = jnp.dot(k * qg, S,
                    preferred_element_type=jnp.float32) * (1.0 / math.sqrt(DV))
        u_s[pl.ds(t * TN, TN), :] = u
        valid = batch_col != B
        u_m = jnp.where(valid, u, NEG)
        tmax = jnp.max(u_m, axis=0, keepdims=True)
        umax_s[:, :] = jnp.maximum(umax_s[:, :], tmax)

    @pl.when(i >= T)
    def _phase2():
        v = jnp.dot(x_ref[:, :], wv_s[:, :],
                    preferred_element_type=jnp.float32)
        u = u_s[pl.ds(t * TN, TN), :]
        valid = batch_col != B
        e = jnp.where(valid, jnp.exp(u - umax_s[:, :]), 0.0)
        e_exp = jnp.dot(e, S.T, preferred_element_type=jnp.float32)
        w_ref[:, :] = e_exp * v
        contract0 = (((0,), (0,)), ((), ()))
        segsum_s[:, :] += lax.dot_general(
            maskf, e, contract0, preferred_element_type=jnp.float32)

        @pl.when(i == 2 * T - 1)
        def _emit():
            ssum_ref[:, :] = segsum_s[:, :]
            qflat_ref[:, :] = qflat_s[:, :]


@jax.jit
def _tc_main(x, batch2d, wk, wv, context, wq):
    return pl.pallas_call(
        _tc_main_body,
        grid=(2 * T,),
        in_specs=[
            pl.BlockSpec((TN, DC), lambda i: (lax.rem(i, T), 0)),
            pl.BlockSpec((TN, 1), lambda i: (lax.rem(i, T), 0)),
            pl.BlockSpec((H, DC, DV), lambda i: (0, 0, 0)),
            pl.BlockSpec((H, DC, DV), lambda i: (0, 0, 0)),
            pl.BlockSpec((B, DC), lambda i: (0, 0)),
            pl.BlockSpec((H, DC, DV), lambda i: (0, 0, 0)),
        ],
        out_specs=[
            pl.BlockSpec((TN, HD), lambda i: (jnp.maximum(i - T, 0), 0)),
            pl.BlockSpec((B, H), lambda i: (0, 0)),
            pl.BlockSpec((B, HD), lambda i: (0, 0)),
        ],
        out_shape=[
            jax.ShapeDtypeStruct((NP, HD), jnp.float32),
            jax.ShapeDtypeStruct((B, H), jnp.float32),
            jax.ShapeDtypeStruct((B, HD), jnp.float32),
        ],
        scratch_shapes=[
            pltpu.VMEM((NP, H), jnp.float32),
            pltpu.VMEM((1, H), jnp.float32),
            pltpu.VMEM((B, H), jnp.float32),
            pltpu.VMEM((B, HD), jnp.float32),
            pltpu.VMEM((DC, HD), jnp.float32),
            pltpu.VMEM((DC, HD), jnp.float32),
        ],
    )(x, batch2d, wk, wv, context, wq)


def _sc_scatter_body(w_hbm, batch_hbm, zeros_hbm, out_hbm, idx_v, w_v, acc_sh,
                     sem):
    c = lax.axis_index("c")
    s = lax.axis_index("s")
    wid = s * NSC + c
    base = wid * RPW

    copies = [
        pltpu.async_copy(batch_hbm.at[pl.ds(base + j * CHUNK, CHUNK)],
                         idx_v.at[j], sem)
        for j in range(NCHUNK)
    ]
    copies.append(pltpu.async_copy(w_hbm.at[pl.ds(base, RPW)], w_v, sem))

    @pl.when(s == 0)
    def _init():
        pltpu.sync_copy(zeros_hbm, acc_sh)
    plsc.subcore_barrier()

    for cp in copies:
        cp.wait()
    for j in range(NCHUNK):
        pltpu.sync_copy(w_v.at[pl.ds(j * CHUNK, CHUNK)],
                        acc_sh.at[idx_v.at[j]], add=True)

    plsc.subcore_barrier()

    @pl.when(s == 0)
    def _emit():
        pltpu.sync_copy(acc_sh, out_hbm.at[c])


@functools.cache
def _sc_scatter():
    mesh = plsc.VectorSubcoreMesh(core_axis_name="c", subcore_axis_name="s",
                                  num_cores=NSC, num_subcores=NSUB)
    return pl.kernel(
        _sc_scatter_body,
        out_type=jax.ShapeDtypeStruct((NSC, ACCR, HD), jnp.float32),
        mesh=mesh,
        scratch_types=[
            pltpu.VMEM((NCHUNK, CHUNK), jnp.int32),
            pltpu.VMEM((RPW, HD), jnp.float32),
            pltpu.VMEM_SHARED((ACCR, HD), jnp.float32),
            pltpu.SemaphoreType.DMA,
        ],
    )


def _tc_finish_body(acc_ref, ssum_ref, qflat_ref, qc_ref, pf_ref, out_ref):
    S = _head_sum_matrix()
    lane2 = lax.broadcasted_iota(jnp.int32, (HD, DV), 0)
    vpos = lax.broadcasted_iota(jnp.int32, (HD, DV), 1)
    R = (lax.rem(lane2, DV) == vpos).astype(jnp.float32)

    acc = acc_ref[0, :, :] + acc_ref[1, :, :]
    agg_raw = acc[0:B, :]
    ssum_exp = jnp.dot(ssum_ref[:, :], S.T,
                       preferred_element_type=jnp.float32)
    agg = agg_raw / (ssum_exp + 1e-16)
    hf = qc_ref[0, 0] * qflat_ref[:, :] + agg
    hsum = jnp.dot(hf, R, preferred_element_type=jnp.float32)
    out_ref[:, :] = jnp.dot(hsum, pf_ref[:, :],
                            preferred_element_type=jnp.float32)


@jax.jit
def _tc_finish(acc, ssum, qflat, qcb, pf):
    return pl.pallas_call(
        _tc_finish_body,
        out_shape=jax.ShapeDtypeStruct((B, DE), jnp.float32),
    )(acc, ssum, qflat, qcb, pf)


def kernel(x, edge_index, batch, context, proj_query, proj_keys, proj_values,
           query_coef, proj_final):
    batch_p = jnp.pad(batch, (0, NP - N), constant_values=B)
    batch2d = batch_p.reshape(NP, 1)
    qcb = jnp.broadcast_to(query_coef.reshape(1, 1), (8, 128))
    zeros = jnp.zeros((ACCR, HD), jnp.float32)

    w_rows, ssum, qflat = _tc_main(x, batch2d, proj_keys, proj_values,
                                   context, proj_query)
    acc = _sc_scatter()(w_rows, batch_p, zeros)
    out = _tc_finish(acc, ssum, qflat, qcb, proj_final)
    return out


# speedup vs baseline: 1.6297x; 1.0708x over previous
import math
import functools

import jax
import jax.numpy as jnp
from jax import lax
from jax.experimental import pallas as pl
from jax.experimental.pallas import tpu as pltpu
from jax.experimental.pallas import tpu_sc as plsc

N = 10000
B = 64
H = 8
DV = 16
DC = 128
DE = 124
HD = H * DV

NP = 10240
TN = 5120
T = NP // TN
NEG = -1e30

NSC = 2
NSUB = 16
NW = NSC * NSUB
RPW = NP // NW
CHUNK = 80
NCHUNK = RPW // CHUNK
ACCR = 72


def _head_sum_matrix():
    lane = lax.broadcasted_iota(jnp.int32, (HD, H), 0)
    head = lax.broadcasted_iota(jnp.int32, (HD, H), 1)
    return (lane // DV == head).astype(jnp.float32)


def _tc_main_body(x_ref, b_ref, wk_ref, wv_ref, ctx_ref, wq_ref,
                  w_ref, ssum_ref, qflat_ref,
                  u_s, umax_s, segsum_s, qflat_s, wk_s, wv_s):
    i = pl.program_id(0)
    t = lax.rem(i, T)
    S = _head_sum_matrix()

    @pl.when(i == 0)
    def _init():
        for h in range(H):
            sl = pl.ds(h * DV, DV)
            wk_s[:, sl] = wk_ref[h]
            wv_s[:, sl] = wv_ref[h]
            qflat_s[:, sl] = jnp.dot(ctx_ref[:, :], wq_ref[h],
                                     preferred_element_type=jnp.float32)
        umax_s[:, :] = jnp.full((1, H), NEG, jnp.float32)
        segsum_s[:, :] = jnp.zeros((B, H), jnp.float32)

    batch_col = b_ref[:, 0:1]
    iota_b = lax.broadcasted_iota(jnp.int32, (TN, B), 1)
    maskf = (batch_col == iota_b).astype(jnp.float32)

    @pl.when(i < T)
    def _phase1():
        k = jnp.dot(x_ref[:, :], wk_s[:, :],
                    preferred_element_type=jnp.float32)
        qg = jnp.dot(maskf, qflat_s[:, :],
                     preferred_element_type=jnp.float32)
        u = jnp.dot(k * qg, S,
                    preferred_element_type=jnp.float32) * (1.0 / math.sqrt(DV))
        u_s[pl.ds(t * TN, TN), :] = u
        valid = batch_col != B
        u_m = jnp.where(valid, u, NEG)
        tmax = jnp.max(u_m, axis=0, keepdims=True)
        umax_s[:, :] = jnp.maximum(umax_s[:, :], tmax)

    @pl.when(i >= T)
    def _phase2():
        v = jnp.dot(x_ref[:, :], wv_s[:, :],
                    preferred_element_type=jnp.float32)
        u = u_s[pl.ds(t * TN, TN), :]
        valid = batch_col != B
        e = jnp.where(valid, jnp.exp(u - umax_s[:, :]), 0.0)
        e_exp = jnp.dot(e, S.T, preferred_element_type=jnp.float32)
        w_ref[:, :] = e_exp * v
        contract0 = (((0,), (0,)), ((), ()))
        segsum_s[:, :] += lax.dot_general(
            maskf, e, contract0, preferred_element_type=jnp.float32)

        @pl.when(i == 2 * T - 1)
        def _emit():
            ssum_ref[:, :] = segsum_s[:, :]
            qflat_ref[:, :] = qflat_s[:, :]


@jax.jit
def _tc_main(x, batch2d, wk, wv, context, wq):
    return pl.pallas_call(
        _tc_main_body,
        grid=(2 * T,),
        in_specs=[
            pl.BlockSpec((TN, DC), lambda i: (lax.rem(i, T), 0)),
            pl.BlockSpec((TN, 1), lambda i: (lax.rem(i, T), 0)),
            pl.BlockSpec((H, DC, DV), lambda i: (0, 0, 0)),
            pl.BlockSpec((H, DC, DV), lambda i: (0, 0, 0)),
            pl.BlockSpec((B, DC), lambda i: (0, 0)),
            pl.BlockSpec((H, DC, DV), lambda i: (0, 0, 0)),
        ],
        out_specs=[
            pl.BlockSpec((TN, HD), lambda i: (jnp.maximum(i - T, 0), 0)),
            pl.BlockSpec((B, H), lambda i: (0, 0)),
            pl.BlockSpec((B, HD), lambda i: (0, 0)),
        ],
        out_shape=[
            jax.ShapeDtypeStruct((NP, HD), jnp.float32),
            jax.ShapeDtypeStruct((B, H), jnp.float32),
            jax.ShapeDtypeStruct((B, HD), jnp.float32),
        ],
        scratch_shapes=[
            pltpu.VMEM((NP, H), jnp.float32),
            pltpu.VMEM((1, H), jnp.float32),
            pltpu.VMEM((B, H), jnp.float32),
            pltpu.VMEM((B, HD), jnp.float32),
            pltpu.VMEM((DC, HD), jnp.float32),
            pltpu.VMEM((DC, HD), jnp.float32),
        ],
    )(x, batch2d, wk, wv, context, wq)


def _sc_scatter_body(w_hbm, batch_hbm, zeros_hbm, out_hbm, idx_v, w_v, acc_sh,
                     sem):
    c = lax.axis_index("c")
    s = lax.axis_index("s")
    wid = s * NSC + c
    base = wid * RPW

    copies = [
        pltpu.async_copy(batch_hbm.at[pl.ds(base + j * CHUNK, CHUNK)],
                         idx_v.at[j], sem)
        for j in range(NCHUNK)
    ]
    copies.append(pltpu.async_copy(w_hbm.at[pl.ds(base, RPW)], w_v, sem))

    @pl.when(s == 0)
    def _init():
        pltpu.sync_copy(zeros_hbm, acc_sh)
    plsc.subcore_barrier()

    for cp in copies:
        cp.wait()
    for j in range(NCHUNK):
        pltpu.sync_copy(w_v.at[pl.ds(j * CHUNK, CHUNK)],
                        acc_sh.at[idx_v.at[j]], add=True)

    plsc.subcore_barrier()

    @pl.when(s == 0)
    def _emit():
        pltpu.sync_copy(acc_sh, out_hbm.at[c])


@functools.cache
def _sc_scatter():
    mesh = plsc.VectorSubcoreMesh(core_axis_name="c", subcore_axis_name="s",
                                  num_cores=NSC, num_subcores=NSUB)
    return pl.kernel(
        _sc_scatter_body,
        out_type=jax.ShapeDtypeStruct((NSC, ACCR, HD), jnp.float32),
        mesh=mesh,
        scratch_types=[
            pltpu.VMEM((NCHUNK, CHUNK), jnp.int32),
            pltpu.VMEM((RPW, HD), jnp.float32),
            pltpu.VMEM_SHARED((ACCR, HD), jnp.float32),
            pltpu.SemaphoreType.DMA,
        ],
    )


def _tc_finish_body(acc_ref, ssum_ref, qflat_ref, qc_ref, pf_ref, out_ref):
    S = _head_sum_matrix()
    lane2 = lax.broadcasted_iota(jnp.int32, (HD, DV), 0)
    vpos = lax.broadcasted_iota(jnp.int32, (HD, DV), 1)
    R = (lax.rem(lane2, DV) == vpos).astype(jnp.float32)

    acc = acc_ref[0, :, :] + acc_ref[1, :, :]
    agg_raw = acc[0:B, :]
    ssum_exp = jnp.dot(ssum_ref[:, :], S.T,
                       preferred_element_type=jnp.float32)
    agg = agg_raw / (ssum_exp + 1e-16)
    hf = qc_ref[0, 0] * qflat_ref[:, :] + agg
    hsum = jnp.dot(hf, R, preferred_element_type=jnp.float32)
    out_ref[:, :] = jnp.dot(hsum, pf_ref[:, :],
                            preferred_element_type=jnp.float32)


@jax.jit
def _tc_finish(acc, ssum, qflat, qcb, pf):
    return pl.pallas_call(
        _tc_finish_body,
        out_shape=jax.ShapeDtypeStruct((B, DE), jnp.float32),
    )(acc, ssum, qflat, qcb, pf)


def kernel(x, edge_index, batch, context, proj_query, proj_keys, proj_values,
           query_coef, proj_final):
    batch_p = jnp.pad(batch, (0, NP - N), constant_values=B)
    batch2d = batch_p.reshape(NP, 1)
    qcb = jnp.broadcast_to(query_coef.reshape(1, 1), (8, 128))
    zeros = jnp.zeros((ACCR, HD), jnp.float32)

    w_rows, ssum, qflat = _tc_main(x, batch2d, proj_keys, proj_values,
                                   context, proj_query)
    acc = _sc_scatter()(w_rows, batch_p, zeros)
    out = _tc_finish(acc, ssum, qflat, qcb, proj_final)
    return out
